# Initial kernel scaffold; baseline (speedup 1.0000x reference)
import functools

import jax
import jax.numpy as jnp
from jax import lax
from jax.experimental import pallas as pl
from jax.experimental.pallas import tpu as pltpu
from jax.experimental.pallas import tpu_sc as plsc

F32 = jnp.float32
_MESH = plsc.VectorSubcoreMesh(core_axis_name="c", subcore_axis_name="s")
_NSUB = 16
_NCORE = 2
_LANE = 128


def _row_partition(nrows, nworkers, w):
    per = nrows // nworkers
    extra = nrows % nworkers
    base = w * per + jnp.minimum(w, extra)
    count = per + jnp.where(w < extra, 1, 0)
    return base, count


def _hist_kernel(nn, pad, r):
    def body(u2d, p2d, zeros_hbm, cnt_u, cnt_p, idx_v, ones_v, acc):
        c = lax.axis_index("c")
        s = lax.axis_index("s")
        for i in range(8):
            ones_v[pl.ds(i * 16, 16)] = jnp.ones((16,), F32)
        chunk = pad // _NSUB
        pltpu.sync_copy(zeros_hbm.at[pl.ds(s * chunk, chunk)],
                        acc.at[pl.ds(s * chunk, chunk)])
        plsc.subcore_barrier()
        base, count = _row_partition(r, _NSUB, s)

        def step(i, carry):
            rr = base + i

            @pl.when(c == 0)
            def _():
                pltpu.sync_copy(u2d.at[pl.ds(rr, 1)], idx_v)
                pltpu.sync_copy(ones_v, acc.at[idx_v.at[0]], add=True)

            @pl.when(c == 1)
            def _():
                pltpu.sync_copy(p2d.at[pl.ds(rr, 1)], idx_v)
                pltpu.sync_copy(ones_v, acc.at[idx_v.at[0]], add=True)

            return carry

        lax.fori_loop(0, count, step, 0)
        plsc.subcore_barrier()

        @pl.when(s == 0)
        def _():
            @pl.when(c == 0)
            def _():
                pltpu.sync_copy(acc.at[pl.ds(0, nn)], cnt_u)

            @pl.when(c == 1)
            def _():
                pltpu.sync_copy(acc.at[pl.ds(0, nn)], cnt_p)

    return body


def _run_hist(u2d, p2d, nn):
    r = u2d.shape[0]
    pad = ((nn + 127) // 128) * 128
    zeros = jnp.zeros((pad,), F32)
    return pl.kernel(
        _hist_kernel(nn, pad, r),
        out_type=[jax.ShapeDtypeStruct((nn,), F32),
                  jax.ShapeDtypeStruct((nn,), F32)],
        mesh=_MESH,
        scratch_types=[pltpu.VMEM((1, _LANE), jnp.int32),
                       pltpu.VMEM((_LANE,), F32),
                       pltpu.VMEM_SHARED((pad,), F32)],
    )(u2d, p2d, zeros)


def _conv_kernel(nn, d, r):
    chunk = nn // _NSUB
    rem = nn % _NSUB

    def blockcopy(s, src, dst):
        pltpu.sync_copy(src.at[pl.ds(s * chunk, chunk)],
                        dst.at[pl.ds(s * chunk, chunk)])
        if rem:
            @pl.when(s == 0)
            def _():
                pltpu.sync_copy(src.at[pl.ds(_NSUB * chunk, rem)],
                                dst.at[pl.ds(_NSUB * chunk, rem)])

    def body(u2d, p2d, g_u, g_p, acc_u_out, acc_p_out,
             idx_u, idx_p, rows, acc, sem):
        c = lax.axis_index("c")
        s = lax.axis_index("s")

        @pl.when(c == 0)
        def _():
            blockcopy(s, g_u, acc)

        @pl.when(c == 1)
        def _():
            blockcopy(s, g_p, acc)

        plsc.subcore_barrier()
        base, count = _row_partition(r, _NSUB, s)

        def step(i, carry):
            rr = base + i
            pltpu.sync_copy(u2d.at[pl.ds(rr, 1)], idx_u)
            pltpu.sync_copy(p2d.at[pl.ds(rr, 1)], idx_p)

            @pl.when(c == 0)
            def _():
                pltpu.async_copy(g_p.at[idx_p.at[0]], rows, sem).wait()
                pltpu.sync_copy(rows, acc.at[idx_u.at[0]], add=True)

            @pl.when(c == 1)
            def _():
                pltpu.async_copy(g_u.at[idx_u.at[0]], rows, sem).wait()
                pltpu.sync_copy(rows, acc.at[idx_p.at[0]], add=True)

            return carry

        lax.fori_loop(0, count, step, 0)
        plsc.subcore_barrier()

        @pl.when(c == 0)
        def _():
            blockcopy(s, acc, acc_u_out)

        @pl.when(c == 1)
        def _():
            blockcopy(s, acc, acc_p_out)

    return body


def _run_conv(u2d, p2d, g_u, g_p):
    nn, d = g_u.shape
    r = u2d.shape[0]
    return pl.kernel(
        _conv_kernel(nn, d, r),
        out_type=[jax.ShapeDtypeStruct((nn, d), F32),
                  jax.ShapeDtypeStruct((nn, d), F32)],
        mesh=_MESH,
        scratch_types=[pltpu.VMEM((1, _LANE), jnp.int32),
                       pltpu.VMEM((1, _LANE), jnp.int32),
                       pltpu.VMEM((_LANE, d), F32),
                       pltpu.VMEM_SHARED((nn, d), F32),
                       pltpu.SemaphoreType.DMA],
    )(u2d, p2d, g_u, g_p)


def _pred_kernel(d, r):
    def body(u2d, p2d, ua, pb, w2_hbm, bp2_hbm, pred,
             idx_u, idx_p, abuf, bbuf, wbuf, bpbuf, obuf, sem):
        c = lax.axis_index("c")
        s = lax.axis_index("s")
        pltpu.sync_copy(w2_hbm, wbuf)
        pltpu.sync_copy(bp2_hbm, bpbuf)
        wv = [wbuf[pl.ds(k * 16, 16)] for k in range(d // 16)]
        bp2 = bpbuf[0]
        w = s * _NCORE + c
        base, count = _row_partition(r, _NSUB * _NCORE, w)

        def step(i, carry):
            rr = base + i
            pltpu.sync_copy(u2d.at[pl.ds(rr, 1)], idx_u)
            pltpu.sync_copy(p2d.at[pl.ds(rr, 1)], idx_p)
            cp_a = pltpu.async_copy(ua.at[idx_u.at[0]], abuf, sem)
            cp_b = pltpu.async_copy(pb.at[idx_p.at[0]], bbuf, sem)
            cp_a.wait()
            cp_b.wait()

            def edge(j, carry2):
                t = None
                for k in range(d // 16):
                    a = abuf[j, pl.ds(k * 16, 16)]
                    b = bbuf[j, pl.ds(k * 16, 16)]
                    part = jnp.maximum(a + b, 0.0) * wv[k]
                    t = part if t is None else t + part
                obuf[j] = jnp.sum(t) + bp2
                return carry2

            lax.fori_loop(0, _LANE, edge, 0)
            pltpu.sync_copy(obuf, pred.at[pl.ds(rr * _LANE, _LANE)])
            return carry

        lax.fori_loop(0, count, step, 0)

    return body


def _run_pred(u2d, p2d, ua, pb, w2, bp2):
    nn, d = ua.shape
    r = u2d.shape[0]
    e = r * _LANE
    bp2_16 = jnp.broadcast_to(bp2.reshape(1), (16,)).astype(F32)
    return pl.kernel(
        _pred_kernel(d, r),
        out_type=jax.ShapeDtypeStruct((e,), F32),
        mesh=_MESH,
        scratch_types=[pltpu.VMEM((1, _LANE), jnp.int32),
                       pltpu.VMEM((1, _LANE), jnp.int32),
                       pltpu.VMEM((_LANE, d), F32),
                       pltpu.VMEM((_LANE, d), F32),
                       pltpu.VMEM((d,), F32),
                       pltpu.VMEM((16,), F32),
                       pltpu.VMEM((_LANE,), F32),
                       pltpu.SemaphoreType.DMA],
    )(u2d, p2d, ua, pb, w2, bp2_16)


def _dot_t(a, w):
    return lax.dot_general(a, w, (((1,), (1,)), ((), ())),
                           precision=lax.Precision.HIGHEST,
                           preferred_element_type=F32)


def _tc_call(fn, n_out, blk, nn, d, args, specs):
    grid = nn // blk
    return pl.pallas_call(
        fn,
        grid=(grid,),
        in_specs=specs,
        out_specs=[pl.BlockSpec((blk, d), lambda i: (i, 0))] * n_out,
        out_shape=[jax.ShapeDtypeStruct((nn, d), F32)] * n_out,
    )(*args)


def _rows_spec(blk, ncol):
    return pl.BlockSpec((blk, ncol), lambda i: (i, 0))


def _full_spec(shape):
    return pl.BlockSpec(shape, lambda i: tuple(0 for _ in shape))


def _tcA(uf, pf, ue, pe, W_uf, b_uf, W_pf, b_pf, W1, blk):
    nn, d = ue.shape

    def fn(uf_r, pf_r, ue_r, pe_r, wuf_r, buf_r, wpf_r, bpf_r, w1_r,
           h1u_r, h1p_r):
        xu = _dot_t(uf_r[...], wuf_r[...]) + buf_r[...] + ue_r[...]
        xp = _dot_t(pf_r[...], wpf_r[...]) + bpf_r[...] + pe_r[...]
        h1u_r[...] = _dot_t(xu, w1_r[...])
        h1p_r[...] = _dot_t(xp, w1_r[...])

    specs = [_rows_spec(blk, uf.shape[1]), _rows_spec(blk, pf.shape[1]),
             _rows_spec(blk, d), _rows_spec(blk, d),
             _full_spec(W_uf.shape), _full_spec((1, d)),
             _full_spec(W_pf.shape), _full_spec((1, d)),
             _full_spec(W1.shape)]
    args = (uf, pf, ue, pe, W_uf, b_uf.reshape(1, d), W_pf,
            b_pf.reshape(1, d), W1)
    return _tc_call(fn, 2, blk, nn, d, args, specs)


def _tcB(cnt_u, cnt_p, h1u, h1p, blk):
    nn, d = h1u.shape

    def fn(cu_r, cp_r, hu_r, hp_r, du_r, dp_r, gu_r, gp_r):
        du = (cu_r[...] + 1.0) ** -0.5
        dp = (cp_r[...] + 1.0) ** -0.5
        du_r[...] = du
        dp_r[...] = dp
        gu_r[...] = du * hu_r[...]
        gp_r[...] = dp * hp_r[...]

    grid = nn // blk
    specs = [_rows_spec(blk, 1), _rows_spec(blk, 1),
             _rows_spec(blk, d), _rows_spec(blk, d)]
    return pl.pallas_call(
        fn,
        grid=(grid,),
        in_specs=specs,
        out_specs=[pl.BlockSpec((blk, 1), lambda i: (i, 0))] * 2 +
                  [pl.BlockSpec((blk, d), lambda i: (i, 0))] * 2,
        out_shape=[jax.ShapeDtypeStruct((nn, 1), F32)] * 2 +
                  [jax.ShapeDtypeStruct((nn, d), F32)] * 2,
    )(cnt_u.reshape(nn, 1), cnt_p.reshape(nn, 1), h1u, h1p)


def _tcC(acc1u, acc1p, dinvu, dinvp, b1, W2, blk):
    nn, d = acc1u.shape

    def fn(au_r, ap_r, du_r, dp_r, b1_r, w2_r, gu_r, gp_r):
        x1u = jnp.maximum(du_r[...] * au_r[...] + b1_r[...], 0.0)
        x1p = jnp.maximum(dp_r[...] * ap_r[...] + b1_r[...], 0.0)
        gu_r[...] = du_r[...] * _dot_t(x1u, w2_r[...])
        gp_r[...] = dp_r[...] * _dot_t(x1p, w2_r[...])

    specs = [_rows_spec(blk, d), _rows_spec(blk, d),
             _rows_spec(blk, 1), _rows_spec(blk, 1),
             _full_spec((1, d)), _full_spec(W2.shape)]
    args = (acc1u, acc1p, dinvu, dinvp, b1.reshape(1, d), W2)
    return _tc_call(fn, 2, blk, nn, d, args, specs)


def _tcD(acc2u, acc2p, dinvu, dinvp, b2, Wp1a, Wp1b, bp1, blk):
    nn, d = acc2u.shape

    def fn(au_r, ap_r, du_r, dp_r, b2_r, wa_r, wb_r, bp1_r, ua_r, pb_r):
        x2u = du_r[...] * au_r[...] + b2_r[...]
        x2p = dp_r[...] * ap_r[...] + b2_r[...]
        ua_r[...] = _dot_t(x2u, wa_r[...]) + bp1_r[...]
        pb_r[...] = _dot_t(x2p, wb_r[...])

    specs = [_rows_spec(blk, d), _rows_spec(blk, d),
             _rows_spec(blk, 1), _rows_spec(blk, 1),
             _full_spec((1, d)), _full_spec(Wp1a.shape),
             _full_spec(Wp1b.shape), _full_spec((1, d))]
    args = (acc2u, acc2p, dinvu, dinvp, b2.reshape(1, d), Wp1a, Wp1b,
            bp1.reshape(1, d))
    return _tc_call(fn, 2, blk, nn, d, args, specs)


def kernel(edge_index, user_features, product_features, user_emb, product_emb,
           W_uf, b_uf, W_pf, b_pf, W1, b1, W2, b2, Wp1, bp1, Wp2, bp2):
    nn = user_features.shape[0]
    d = W1.shape[0]
    e = edge_index.shape[1]
    r = e // _LANE
    blk = 2500

    u2d = edge_index[0].reshape(r, _LANE)
    p2d = edge_index[1].reshape(r, _LANE)

    cnt_u, cnt_p = _run_hist(u2d, p2d, nn)
    h1u, h1p = _tcA(user_features, product_features, user_emb, product_emb,
                    W_uf, b_uf, W_pf, b_pf, W1, blk)
    dinvu, dinvp, g1u, g1p = _tcB(cnt_u, cnt_p, h1u, h1p, blk)
    acc1u, acc1p = _run_conv(u2d, p2d, g1u, g1p)
    g2u, g2p = _tcC(acc1u, acc1p, dinvu, dinvp, b1, W2, blk)
    acc2u, acc2p = _run_conv(u2d, p2d, g2u, g2p)
    ua, pb = _tcD(acc2u, acc2p, dinvu, dinvp, b2,
                  Wp1[:, :d], Wp1[:, d:], bp1, blk)
    pred = _run_pred(u2d, p2d, ua, pb, Wp2.reshape(d), bp2)
    return pred


# speedup vs baseline: 28.1908x; 28.1908x over previous
import functools

import jax
import jax.numpy as jnp
from jax import lax
from jax.experimental import pallas as pl
from jax.experimental.pallas import tpu as pltpu
from jax.experimental.pallas import tpu_sc as plsc

F32 = jnp.float32
_MESH = plsc.VectorSubcoreMesh(core_axis_name="c", subcore_axis_name="s")
_SC_PARAMS = pltpu.CompilerParams(use_tc_tiling_on_sc=False,
                                  needs_layout_passes=False)
_NSUB = 16
_NCORE = 2
_LANE = 128
_CH = 13


def _row_partition(nrows, nworkers, w):
    per = nrows // nworkers
    extra = nrows % nworkers
    base = w * per + jnp.minimum(w, extra)
    count = per + jnp.where(w < extra, 1, 0)
    return base, count


def _hist_kernel(pad, r):
    chunk = pad // _NSUB

    def body(u2d, p2d, cnt_u, cnt_p, idx_v, ones_v, stage, acc, ssem):
        c = lax.axis_index("c")
        s = lax.axis_index("s")
        for i in range(8):
            ones_v[pl.ds(i * 16, 16)] = jnp.ones((16,), F32)
        zero16 = jnp.zeros((16,), F32)
        for i in range(chunk // 16):
            stage[pl.ds(i * 16, 16)] = zero16
        pltpu.sync_copy(stage, acc.at[pl.ds(s * chunk, chunk)])
        plsc.subcore_barrier()
        base, count = _row_partition(r, _NSUB, s)
        nch = count // _CH
        rem = count - nch * _CH

        def run(e2d):
            def chunk_body(jc, carry):
                r0 = base + jc * _CH
                pltpu.sync_copy(e2d.at[pl.ds(r0, _CH)], idx_v)
                descs = [pltpu.async_copy(ones_v, acc.at[idx_v.at[k]],
                                          ssem, add=True)
                         for k in range(_CH)]
                for de in descs:
                    de.wait()
                return carry

            lax.fori_loop(0, nch, chunk_body, 0)

            def tail_body(i, carry):
                rr = base + nch * _CH + i
                pltpu.sync_copy(e2d.at[pl.ds(rr, 1)], idx_v.at[pl.ds(0, 1)])
                pltpu.sync_copy(ones_v, acc.at[idx_v.at[0]], add=True)
                return carry

            lax.fori_loop(0, rem, tail_body, 0)

        @pl.when(c == 0)
        def _():
            run(u2d)

        @pl.when(c == 1)
        def _():
            run(p2d)

        plsc.subcore_barrier()
        pltpu.sync_copy(acc.at[pl.ds(s * chunk, chunk)], stage)

        @pl.when(c == 0)
        def _():
            pltpu.sync_copy(stage, cnt_u.at[pl.ds(s * chunk, chunk)])

        @pl.when(c == 1)
        def _():
            pltpu.sync_copy(stage, cnt_p.at[pl.ds(s * chunk, chunk)])

    return body


def _run_hist(u2d, p2d, nn):
    r = u2d.shape[0]
    pad = ((nn + 127) // 128) * 128
    cu, cp = pl.kernel(
        _hist_kernel(pad, r),
        out_type=[jax.ShapeDtypeStruct((pad,), F32),
                  jax.ShapeDtypeStruct((pad,), F32)],
        mesh=_MESH,
        compiler_params=_SC_PARAMS,
        scratch_types=[pltpu.VMEM((_CH, _LANE), jnp.int32),
                       pltpu.VMEM((_LANE,), F32),
                       pltpu.VMEM((pad // _NSUB,), F32),
                       pltpu.VMEM_SHARED((pad,), F32),
                       pltpu.SemaphoreType.DMA],
    )(u2d, p2d)
    return cu[:nn], cp[:nn]


_SUBROWS = 112
_NBUF = 3
_LOOK = 1


def _conv_kernel(pad, d, r):
    chunk = pad // _NSUB
    nit = chunk // _SUBROWS

    def body(u2d, p2d, g_u, g_p, acc_u_out, acc_p_out,
             idxu_blk, idxp_blk, rows0, rows1, rows2, acc,
             gsem0, gsem1, gsem2, ssem0, ssem1, ssem2, isem0, isem1):
        c = lax.axis_index("c")
        s = lax.axis_index("s")
        rows = (rows0, rows1, rows2)
        gsem = (gsem0, gsem1, gsem2)
        ssem = (ssem0, ssem1, ssem2)

        def blockcopy(src, dst):
            via = rows0.at[pl.ds(0, _SUBROWS)]
            for t in range(nit):
                off = s * chunk + t * _SUBROWS
                pltpu.sync_copy(src.at[pl.ds(off, _SUBROWS)], via)
                pltpu.sync_copy(via, dst.at[pl.ds(off, _SUBROWS)])

        base, count = _row_partition(r, _NSUB, s)
        nch = count // _CH
        rem = count - nch * _CH

        def run(g_self, g_src, idx_src, idx_dst, out_ref):
            blockcopy(g_self, acc)
            plsc.subcore_barrier()

            def chunk_body(jc, carry):
                r0 = base + jc * _CH
                ci = pltpu.async_copy(u2d.at[pl.ds(r0, _CH)], idxu_blk, isem0)
                cj = pltpu.async_copy(p2d.at[pl.ds(r0, _CH)], idxp_blk, isem1)
                ci.wait()
                cj.wait()
                g_desc = {}
                s_desc = {}
                for j in range(_LOOK):
                    g_desc[j] = pltpu.async_copy(
                        g_src.at[idx_src.at[j]], rows[j % _NBUF],
                        gsem[j % _NBUF])
                for k in range(_CH):
                    b = k % _NBUF
                    j = k + _LOOK
                    if j < _CH:
                        if j - _NBUF >= 0:
                            s_desc[j - _NBUF].wait()
                        g_desc[j] = pltpu.async_copy(
                            g_src.at[idx_src.at[j]], rows[j % _NBUF],
                            gsem[j % _NBUF])
                    g_desc[k].wait()
                    s_desc[k] = pltpu.async_copy(
                        rows[b], acc.at[idx_dst.at[k]], ssem[b], add=True)
                for k in range(max(0, _CH - _NBUF), _CH):
                    s_desc[k].wait()
                return carry

            lax.fori_loop(0, nch, chunk_body, 0)

            def tail_body(i, carry):
                rr = base + nch * _CH + i
                pltpu.sync_copy(u2d.at[pl.ds(rr, 1)], idxu_blk.at[pl.ds(0, 1)])
                pltpu.sync_copy(p2d.at[pl.ds(rr, 1)], idxp_blk.at[pl.ds(0, 1)])
                pltpu.async_copy(g_src.at[idx_src.at[0]], rows[0],
                                 gsem[0]).wait()
                pltpu.sync_copy(rows[0], acc.at[idx_dst.at[0]], add=True)
                return carry

            lax.fori_loop(0, rem, tail_body, 0)
            plsc.subcore_barrier()
            blockcopy(acc, out_ref)

        @pl.when(c == 0)
        def _():
            run(g_u, g_p, idxp_blk, idxu_blk, acc_u_out)

        @pl.when(c == 1)
        def _():
            run(g_p, g_u, idxu_blk, idxp_blk, acc_p_out)

    return body


def _run_conv(u2d, p2d, g_u, g_p, pad):
    nn, d = g_u.shape
    r = u2d.shape[0]
    gup = jnp.pad(g_u, ((0, pad - nn), (0, 0)))
    gpp = jnp.pad(g_p, ((0, pad - nn), (0, 0)))
    au, ap = pl.kernel(
        _conv_kernel(pad, d, r),
        out_type=[jax.ShapeDtypeStruct((pad, d), F32),
                  jax.ShapeDtypeStruct((pad, d), F32)],
        mesh=_MESH,
        compiler_params=_SC_PARAMS,
        scratch_types=[pltpu.VMEM((_CH, _LANE), jnp.int32),
                       pltpu.VMEM((_CH, _LANE), jnp.int32)] +
                      [pltpu.VMEM((_LANE, d), F32)] * _NBUF +
                      [pltpu.VMEM_SHARED((pad, d), F32)] +
                      [pltpu.SemaphoreType.DMA] * (2 * _NBUF + 2),
    )(u2d, p2d, gup, gpp)
    return au[:nn], ap[:nn]


def _pred_kernel(d, r):
    nb = d // 16

    def body(u2d, p2d, ua, pb, w2_hbm, bp2_hbm, pred,
             idx_u, idx_p, abuf, bbuf, wbuf, bpbuf, obuf, tbuf, sem):
        c = lax.axis_index("c")
        s = lax.axis_index("s")
        pltpu.sync_copy(w2_hbm, wbuf)
        pltpu.sync_copy(bp2_hbm, bpbuf)
        wv = [wbuf[pl.ds(k * 16, 16)] for k in range(nb)]
        bpv = bpbuf[pl.ds(0, 16)]
        lane = lax.iota(jnp.int32, 16)
        w = s * _NCORE + c
        base, count = _row_partition(r, _NSUB * _NCORE, w)

        def step(i, carry):
            rr = base + i
            pltpu.sync_copy(u2d.at[pl.ds(rr, 1)], idx_u)
            pltpu.sync_copy(p2d.at[pl.ds(rr, 1)], idx_p)
            cp_a = pltpu.async_copy(ua.at[idx_u.at[0]], abuf, sem)
            cp_b = pltpu.async_copy(pb.at[idx_p.at[0]], bbuf, sem)
            cp_a.wait()
            cp_b.wait()

            def group(g, carry2):
                for jj in range(16):
                    j = g * 16 + jj
                    t = None
                    for k in range(nb):
                        a = abuf[j, pl.ds(k * 16, 16)]
                        b = bbuf[j, pl.ds(k * 16, 16)]
                        part = jnp.maximum(a + b, 0.0) * wv[k]
                        t = part if t is None else t + part
                    plsc.store_scatter(tbuf, [lane * 16 + jj], t)
                acc = bpv
                for kk in range(16):
                    acc = acc + tbuf[pl.ds(kk * 16, 16)]
                obuf[pl.ds(g * 16, 16)] = acc
                return carry2

            lax.fori_loop(0, _LANE // 16, group, 0)
            pltpu.sync_copy(obuf, pred.at[pl.ds(rr * _LANE, _LANE)])
            return carry

        lax.fori_loop(0, count, step, 0)

    return body


def _run_pred(u2d, p2d, ua, pb, w2, bp2):
    nn, d = ua.shape
    r = u2d.shape[0]
    e = r * _LANE
    bp2_16 = jnp.broadcast_to(bp2.reshape(1), (16,)).astype(F32)
    return pl.kernel(
        _pred_kernel(d, r),
        out_type=jax.ShapeDtypeStruct((e,), F32),
        mesh=_MESH,
        compiler_params=_SC_PARAMS,
        scratch_types=[pltpu.VMEM((1, _LANE), jnp.int32),
                       pltpu.VMEM((1, _LANE), jnp.int32),
                       pltpu.VMEM((_LANE, d), F32),
                       pltpu.VMEM((_LANE, d), F32),
                       pltpu.VMEM((d,), F32),
                       pltpu.VMEM((16,), F32),
                       pltpu.VMEM((_LANE,), F32),
                       pltpu.VMEM((256,), F32),
                       pltpu.SemaphoreType.DMA],
    )(u2d, p2d, ua, pb, w2, bp2_16)


def _dot_t(a, w):
    return lax.dot_general(a, w, (((1,), (1,)), ((), ())),
                           precision=lax.Precision.HIGHEST,
                           preferred_element_type=F32)


def _tc_call(fn, n_out, blk, nn, d, args, specs):
    grid = nn // blk
    return pl.pallas_call(
        fn,
        grid=(grid,),
        in_specs=specs,
        out_specs=[pl.BlockSpec((blk, d), lambda i: (i, 0))] * n_out,
        out_shape=[jax.ShapeDtypeStruct((nn, d), F32)] * n_out,
    )(*args)


def _rows_spec(blk, ncol):
    return pl.BlockSpec((blk, ncol), lambda i: (i, 0))


def _full_spec(shape):
    return pl.BlockSpec(shape, lambda i: tuple(0 for _ in shape))


def _tcA(uf, pf, ue, pe, W_uf, b_uf, W_pf, b_pf, W1, blk):
    nn, d = ue.shape

    def fn(uf_r, pf_r, ue_r, pe_r, wuf_r, buf_r, wpf_r, bpf_r, w1_r,
           h1u_r, h1p_r):
        xu = _dot_t(uf_r[...], wuf_r[...]) + buf_r[...] + ue_r[...]
        xp = _dot_t(pf_r[...], wpf_r[...]) + bpf_r[...] + pe_r[...]
        h1u_r[...] = _dot_t(xu, w1_r[...])
        h1p_r[...] = _dot_t(xp, w1_r[...])

    specs = [_rows_spec(blk, uf.shape[1]), _rows_spec(blk, pf.shape[1]),
             _rows_spec(blk, d), _rows_spec(blk, d),
             _full_spec(W_uf.shape), _full_spec((1, d)),
             _full_spec(W_pf.shape), _full_spec((1, d)),
             _full_spec(W1.shape)]
    args = (uf, pf, ue, pe, W_uf, b_uf.reshape(1, d), W_pf,
            b_pf.reshape(1, d), W1)
    return _tc_call(fn, 2, blk, nn, d, args, specs)


def _tcB(cnt_u, cnt_p, h1u, h1p, blk):
    nn, d = h1u.shape

    def fn(cu_r, cp_r, hu_r, hp_r, du_r, dp_r, gu_r, gp_r):
        du = (cu_r[...] + 1.0) ** -0.5
        dp = (cp_r[...] + 1.0) ** -0.5
        du_r[...] = du
        dp_r[...] = dp
        gu_r[...] = du * hu_r[...]
        gp_r[...] = dp * hp_r[...]

    grid = nn // blk
    specs = [_rows_spec(blk, 1), _rows_spec(blk, 1),
             _rows_spec(blk, d), _rows_spec(blk, d)]
    return pl.pallas_call(
        fn,
        grid=(grid,),
        in_specs=specs,
        out_specs=[pl.BlockSpec((blk, 1), lambda i: (i, 0))] * 2 +
                  [pl.BlockSpec((blk, d), lambda i: (i, 0))] * 2,
        out_shape=[jax.ShapeDtypeStruct((nn, 1), F32)] * 2 +
                  [jax.ShapeDtypeStruct((nn, d), F32)] * 2,
    )(cnt_u.reshape(nn, 1), cnt_p.reshape(nn, 1), h1u, h1p)


def _tcC(acc1u, acc1p, dinvu, dinvp, b1, W2, blk):
    nn, d = acc1u.shape

    def fn(au_r, ap_r, du_r, dp_r, b1_r, w2_r, gu_r, gp_r):
        x1u = jnp.maximum(du_r[...] * au_r[...] + b1_r[...], 0.0)
        x1p = jnp.maximum(dp_r[...] * ap_r[...] + b1_r[...], 0.0)
        gu_r[...] = du_r[...] * _dot_t(x1u, w2_r[...])
        gp_r[...] = dp_r[...] * _dot_t(x1p, w2_r[...])

    specs = [_rows_spec(blk, d), _rows_spec(blk, d),
             _rows_spec(blk, 1), _rows_spec(blk, 1),
             _full_spec((1, d)), _full_spec(W2.shape)]
    args = (acc1u, acc1p, dinvu, dinvp, b1.reshape(1, d), W2)
    return _tc_call(fn, 2, blk, nn, d, args, specs)


def _tcD(acc2u, acc2p, dinvu, dinvp, b2, Wp1a, Wp1b, bp1, blk):
    nn, d = acc2u.shape

    def fn(au_r, ap_r, du_r, dp_r, b2_r, wa_r, wb_r, bp1_r, ua_r, pb_r):
        x2u = du_r[...] * au_r[...] + b2_r[...]
        x2p = dp_r[...] * ap_r[...] + b2_r[...]
        ua_r[...] = _dot_t(x2u, wa_r[...]) + bp1_r[...]
        pb_r[...] = _dot_t(x2p, wb_r[...])

    specs = [_rows_spec(blk, d), _rows_spec(blk, d),
             _rows_spec(blk, 1), _rows_spec(blk, 1),
             _full_spec((1, d)), _full_spec(Wp1a.shape),
             _full_spec(Wp1b.shape), _full_spec((1, d))]
    args = (acc2u, acc2p, dinvu, dinvp, b2.reshape(1, d), Wp1a, Wp1b,
            bp1.reshape(1, d))
    return _tc_call(fn, 2, blk, nn, d, args, specs)


def kernel(edge_index, user_features, product_features, user_emb, product_emb,
           W_uf, b_uf, W_pf, b_pf, W1, b1, W2, b2, Wp1, bp1, Wp2, bp2):
    nn = user_features.shape[0]
    d = W1.shape[0]
    e = edge_index.shape[1]
    r = e // _LANE
    blk = 5000

    u2d = edge_index[0].reshape(r, _LANE)
    p2d = edge_index[1].reshape(r, _LANE)

    pad = ((nn + 127) // 128) * 128
    cnt_u, cnt_p = _run_hist(u2d, p2d, nn)
    h1u, h1p = _tcA(user_features, product_features, user_emb, product_emb,
                    W_uf, b_uf, W_pf, b_pf, W1, blk)
    dinvu, dinvp, g1u, g1p = _tcB(cnt_u, cnt_p, h1u, h1p, blk)
    acc1u, acc1p = _run_conv(u2d, p2d, g1u, g1p, pad)
    g2u, g2p = _tcC(acc1u, acc1p, dinvu, dinvp, b1, W2, blk)
    acc2u, acc2p = _run_conv(u2d, p2d, g2u, g2p, pad)
    ua, pb = _tcD(acc2u, acc2p, dinvu, dinvp, b2,
                  Wp1[:, :d], Wp1[:, d:], bp1, blk)
    pred = _run_pred(u2d, p2d, ua, pb, Wp2.reshape(d), bp2)
    return pred


# speedup vs baseline: 35.0835x; 1.2445x over previous
import functools

import jax
import jax.numpy as jnp
from jax import lax
from jax.experimental import pallas as pl
from jax.experimental.pallas import tpu as pltpu
from jax.experimental.pallas import tpu_sc as plsc

F32 = jnp.float32
_MESH = plsc.VectorSubcoreMesh(core_axis_name="c", subcore_axis_name="s")
_SC_PARAMS = pltpu.CompilerParams(use_tc_tiling_on_sc=False,
                                  needs_layout_passes=False)
_NSUB = 16
_NCORE = 2
_LANE = 128
_CH = 13


def _row_partition(nrows, nworkers, w):
    per = nrows // nworkers
    extra = nrows % nworkers
    base = w * per + jnp.minimum(w, extra)
    count = per + jnp.where(w < extra, 1, 0)
    return base, count


def _hist_kernel(pad, r):
    chunk = pad // _NSUB

    def body(u2d, p2d, cnt_u, cnt_p, idx_v, ones_v, stage, acc, ssem):
        c = lax.axis_index("c")
        s = lax.axis_index("s")
        for i in range(8):
            ones_v[pl.ds(i * 16, 16)] = jnp.ones((16,), F32)
        zero16 = jnp.zeros((16,), F32)
        for i in range(chunk // 16):
            stage[pl.ds(i * 16, 16)] = zero16
        pltpu.sync_copy(stage, acc.at[pl.ds(s * chunk, chunk)])
        plsc.subcore_barrier()
        base, count = _row_partition(r, _NSUB, s)
        nch = count // _CH
        rem = count - nch * _CH

        def run(e2d):
            def chunk_body(jc, carry):
                r0 = base + jc * _CH
                pltpu.sync_copy(e2d.at[pl.ds(r0, _CH)], idx_v)
                descs = [pltpu.async_copy(ones_v, acc.at[idx_v.at[k]],
                                          ssem, add=True)
                         for k in range(_CH)]
                for de in descs:
                    de.wait()
                return carry

            lax.fori_loop(0, nch, chunk_body, 0)

            def tail_body(i, carry):
                rr = base + nch * _CH + i
                pltpu.sync_copy(e2d.at[pl.ds(rr, 1)], idx_v.at[pl.ds(0, 1)])
                pltpu.sync_copy(ones_v, acc.at[idx_v.at[0]], add=True)
                return carry

            lax.fori_loop(0, rem, tail_body, 0)

        @pl.when(c == 0)
        def _():
            run(u2d)

        @pl.when(c == 1)
        def _():
            run(p2d)

        plsc.subcore_barrier()
        pltpu.sync_copy(acc.at[pl.ds(s * chunk, chunk)], stage)

        @pl.when(c == 0)
        def _():
            pltpu.sync_copy(stage, cnt_u.at[pl.ds(s * chunk, chunk)])

        @pl.when(c == 1)
        def _():
            pltpu.sync_copy(stage, cnt_p.at[pl.ds(s * chunk, chunk)])

    return body


def _run_hist(u2d, p2d, nn):
    r = u2d.shape[0]
    pad = ((nn + 127) // 128) * 128
    cu, cp = pl.kernel(
        _hist_kernel(pad, r),
        out_type=[jax.ShapeDtypeStruct((pad,), F32),
                  jax.ShapeDtypeStruct((pad,), F32)],
        mesh=_MESH,
        compiler_params=_SC_PARAMS,
        scratch_types=[pltpu.VMEM((_CH, _LANE), jnp.int32),
                       pltpu.VMEM((_LANE,), F32),
                       pltpu.VMEM((pad // _NSUB,), F32),
                       pltpu.VMEM_SHARED((pad,), F32),
                       pltpu.SemaphoreType.DMA],
    )(u2d, p2d)
    return cu[:nn], cp[:nn]


_SUBROWS = 112
_NBUF = 3
_LOOK = 1


def _conv_kernel(pad, d, r):
    chunk = pad // _NSUB
    nit = chunk // _SUBROWS

    def body(u2d, p2d, g_u, g_p, acc_u_out, acc_p_out,
             idxu_blk, idxp_blk, rows0, rows1, rows2, acc,
             gsem0, gsem1, gsem2, ssem0, ssem1, ssem2, isem0, isem1):
        c = lax.axis_index("c")
        s = lax.axis_index("s")
        rows = (rows0, rows1, rows2)
        gsem = (gsem0, gsem1, gsem2)
        ssem = (ssem0, ssem1, ssem2)

        def blockcopy(src, dst):
            via = rows0.at[pl.ds(0, _SUBROWS)]
            for t in range(nit):
                off = s * chunk + t * _SUBROWS
                pltpu.sync_copy(src.at[pl.ds(off, _SUBROWS)], via)
                pltpu.sync_copy(via, dst.at[pl.ds(off, _SUBROWS)])

        base, count = _row_partition(r, _NSUB, s)
        nch = count // _CH
        rem = count - nch * _CH

        def run(g_self, g_src, idx_src, idx_dst, out_ref):
            blockcopy(g_self, acc)
            plsc.subcore_barrier()

            def chunk_body(jc, carry):
                r0 = base + jc * _CH
                ci = pltpu.async_copy(u2d.at[pl.ds(r0, _CH)], idxu_blk, isem0)
                cj = pltpu.async_copy(p2d.at[pl.ds(r0, _CH)], idxp_blk, isem1)
                ci.wait()
                cj.wait()
                g_desc = {}
                s_desc = {}
                for j in range(_LOOK):
                    g_desc[j] = pltpu.async_copy(
                        g_src.at[idx_src.at[j]], rows[j % _NBUF],
                        gsem[j % _NBUF])
                for k in range(_CH):
                    b = k % _NBUF
                    j = k + _LOOK
                    if j < _CH:
                        if j - _NBUF >= 0:
                            s_desc[j - _NBUF].wait()
                        g_desc[j] = pltpu.async_copy(
                            g_src.at[idx_src.at[j]], rows[j % _NBUF],
                            gsem[j % _NBUF])
                    g_desc[k].wait()
                    s_desc[k] = pltpu.async_copy(
                        rows[b], acc.at[idx_dst.at[k]], ssem[b], add=True)
                for k in range(max(0, _CH - _NBUF), _CH):
                    s_desc[k].wait()
                return carry

            lax.fori_loop(0, nch, chunk_body, 0)

            def tail_body(i, carry):
                rr = base + nch * _CH + i
                pltpu.sync_copy(u2d.at[pl.ds(rr, 1)], idxu_blk.at[pl.ds(0, 1)])
                pltpu.sync_copy(p2d.at[pl.ds(rr, 1)], idxp_blk.at[pl.ds(0, 1)])
                pltpu.async_copy(g_src.at[idx_src.at[0]], rows[0],
                                 gsem[0]).wait()
                pltpu.sync_copy(rows[0], acc.at[idx_dst.at[0]], add=True)
                return carry

            lax.fori_loop(0, rem, tail_body, 0)
            plsc.subcore_barrier()
            blockcopy(acc, out_ref)

        @pl.when(c == 0)
        def _():
            run(g_u, g_p, idxp_blk, idxu_blk, acc_u_out)

        @pl.when(c == 1)
        def _():
            run(g_p, g_u, idxu_blk, idxp_blk, acc_p_out)

    return body


def _run_conv(u2d, p2d, g_u, g_p, pad):
    nn, d = g_u.shape
    r = u2d.shape[0]
    gup = jnp.pad(g_u, ((0, pad - nn), (0, 0)))
    gpp = jnp.pad(g_p, ((0, pad - nn), (0, 0)))
    au, ap = pl.kernel(
        _conv_kernel(pad, d, r),
        out_type=[jax.ShapeDtypeStruct((pad, d), F32),
                  jax.ShapeDtypeStruct((pad, d), F32)],
        mesh=_MESH,
        compiler_params=_SC_PARAMS,
        scratch_types=[pltpu.VMEM((_CH, _LANE), jnp.int32),
                       pltpu.VMEM((_CH, _LANE), jnp.int32)] +
                      [pltpu.VMEM((_LANE, d), F32)] * _NBUF +
                      [pltpu.VMEM_SHARED((pad, d), F32)] +
                      [pltpu.SemaphoreType.DMA] * (2 * _NBUF + 2),
    )(u2d, p2d, gup, gpp)
    return au[:nn], ap[:nn]


_PCH = 8


def _pred_kernel(d, r):
    nb = d // 16

    def body(u2d, p2d, ua, pb, w2_hbm, bp2_hbm, pred,
             idx_u, idx_p, abuf0, abuf1, bbuf0, bbuf1, wbuf, bpbuf,
             obuf, tbuf, asem0, asem1, bsem0, bsem1, isem0, isem1):
        c = lax.axis_index("c")
        s = lax.axis_index("s")
        abuf = (abuf0, abuf1)
        bbuf = (bbuf0, bbuf1)
        asem = (asem0, asem1)
        bsem = (bsem0, bsem1)
        pltpu.sync_copy(w2_hbm, wbuf)
        pltpu.sync_copy(bp2_hbm, bpbuf)
        wv = [wbuf[pl.ds(k * 16, 16)] for k in range(nb)]
        bpv = bpbuf[pl.ds(0, 16)]
        lane = lax.iota(jnp.int32, 16)
        w = s * _NCORE + c
        base, count = _row_partition(r, _NSUB * _NCORE, w)
        nch = count // _PCH
        rem = count - nch * _PCH

        def compute_row(ab, bb, k):
            def group(g, carry2):
                for jj in range(16):
                    j = g * 16 + jj
                    t = None
                    for kk in range(nb):
                        a = ab[j, pl.ds(kk * 16, 16)]
                        b = bb[j, pl.ds(kk * 16, 16)]
                        part = jnp.maximum(a + b, 0.0) * wv[kk]
                        t = part if t is None else t + part
                    plsc.store_scatter(tbuf, [lane * 16 + jj], t)
                t8 = [tbuf[pl.ds(kk * 32, 16)] + tbuf[pl.ds(kk * 32 + 16, 16)]
                      for kk in range(8)]
                t4 = [t8[2 * kk] + t8[2 * kk + 1] for kk in range(4)]
                t2 = [t4[0] + t4[1], t4[2] + t4[3]]
                obuf[pl.ds(k * _LANE + g * 16, 16)] = bpv + (t2[0] + t2[1])
                return carry2

            lax.fori_loop(0, _LANE // 16, group, 0)

        def chunk_body(jc, carry):
            r0 = base + jc * _PCH
            ci = pltpu.async_copy(u2d.at[pl.ds(r0, _PCH)], idx_u, isem0)
            cj = pltpu.async_copy(p2d.at[pl.ds(r0, _PCH)], idx_p, isem1)
            ci.wait()
            cj.wait()
            a_desc = {0: pltpu.async_copy(ua.at[idx_u.at[0]], abuf[0],
                                          asem[0])}
            b_desc = {0: pltpu.async_copy(pb.at[idx_p.at[0]], bbuf[0],
                                          bsem[0])}
            for k in range(_PCH):
                b = k & 1
                if k + 1 < _PCH:
                    a_desc[k + 1] = pltpu.async_copy(
                        ua.at[idx_u.at[k + 1]], abuf[1 - b], asem[1 - b])
                    b_desc[k + 1] = pltpu.async_copy(
                        pb.at[idx_p.at[k + 1]], bbuf[1 - b], bsem[1 - b])
                a_desc[k].wait()
                b_desc[k].wait()
                compute_row(abuf[b], bbuf[b], k)
            pltpu.sync_copy(obuf,
                            pred.at[pl.ds(r0 * _LANE, _PCH * _LANE)])
            return carry

        lax.fori_loop(0, nch, chunk_body, 0)

        def tail_body(i, carry):
            rr = base + nch * _PCH + i
            pltpu.sync_copy(u2d.at[pl.ds(rr, 1)], idx_u.at[pl.ds(0, 1)])
            pltpu.sync_copy(p2d.at[pl.ds(rr, 1)], idx_p.at[pl.ds(0, 1)])
            cp_a = pltpu.async_copy(ua.at[idx_u.at[0]], abuf[0], asem[0])
            cp_b = pltpu.async_copy(pb.at[idx_p.at[0]], bbuf[0], bsem[0])
            cp_a.wait()
            cp_b.wait()
            compute_row(abuf[0], bbuf[0], 0)
            pltpu.sync_copy(obuf.at[pl.ds(0, _LANE)],
                            pred.at[pl.ds(rr * _LANE, _LANE)])
            return carry

        lax.fori_loop(0, rem, tail_body, 0)

    return body


def _run_pred(u2d, p2d, ua, pb, w2, bp2):
    nn, d = ua.shape
    r = u2d.shape[0]
    e = r * _LANE
    bp2_16 = jnp.broadcast_to(bp2.reshape(1), (16,)).astype(F32)
    return pl.kernel(
        _pred_kernel(d, r),
        out_type=jax.ShapeDtypeStruct((e,), F32),
        mesh=_MESH,
        compiler_params=_SC_PARAMS,
        scratch_types=[pltpu.VMEM((_PCH, _LANE), jnp.int32),
                       pltpu.VMEM((_PCH, _LANE), jnp.int32)] +
                      [pltpu.VMEM((_LANE, d), F32)] * 4 +
                      [pltpu.VMEM((d,), F32),
                       pltpu.VMEM((16,), F32),
                       pltpu.VMEM((_PCH * _LANE,), F32),
                       pltpu.VMEM((256,), F32)] +
                      [pltpu.SemaphoreType.DMA] * 6,
    )(u2d, p2d, ua, pb, w2, bp2_16)


def _dot_t(a, w):
    return lax.dot_general(a, w, (((1,), (1,)), ((), ())),
                           precision=lax.Precision.HIGHEST,
                           preferred_element_type=F32)


def _tc_call(fn, n_out, blk, nn, d, args, specs):
    grid = nn // blk
    return pl.pallas_call(
        fn,
        grid=(grid,),
        in_specs=specs,
        out_specs=[pl.BlockSpec((blk, d), lambda i: (i, 0))] * n_out,
        out_shape=[jax.ShapeDtypeStruct((nn, d), F32)] * n_out,
    )(*args)


def _rows_spec(blk, ncol):
    return pl.BlockSpec((blk, ncol), lambda i: (i, 0))


def _full_spec(shape):
    return pl.BlockSpec(shape, lambda i: tuple(0 for _ in shape))


def _tcA(uf, pf, ue, pe, W_uf, b_uf, W_pf, b_pf, W1, blk):
    nn, d = ue.shape

    def fn(uf_r, pf_r, ue_r, pe_r, wuf_r, buf_r, wpf_r, bpf_r, w1_r,
           h1u_r, h1p_r):
        xu = _dot_t(uf_r[...], wuf_r[...]) + buf_r[...] + ue_r[...]
        xp = _dot_t(pf_r[...], wpf_r[...]) + bpf_r[...] + pe_r[...]
        h1u_r[...] = _dot_t(xu, w1_r[...])
        h1p_r[...] = _dot_t(xp, w1_r[...])

    specs = [_rows_spec(blk, uf.shape[1]), _rows_spec(blk, pf.shape[1]),
             _rows_spec(blk, d), _rows_spec(blk, d),
             _full_spec(W_uf.shape), _full_spec((1, d)),
             _full_spec(W_pf.shape), _full_spec((1, d)),
             _full_spec(W1.shape)]
    args = (uf, pf, ue, pe, W_uf, b_uf.reshape(1, d), W_pf,
            b_pf.reshape(1, d), W1)
    return _tc_call(fn, 2, blk, nn, d, args, specs)


def _tcB(cnt_u, cnt_p, h1u, h1p, blk):
    nn, d = h1u.shape

    def fn(cu_r, cp_r, hu_r, hp_r, du_r, dp_r, gu_r, gp_r):
        du = (cu_r[...] + 1.0) ** -0.5
        dp = (cp_r[...] + 1.0) ** -0.5
        du_r[...] = du
        dp_r[...] = dp
        gu_r[...] = du * hu_r[...]
        gp_r[...] = dp * hp_r[...]

    grid = nn // blk
    specs = [_rows_spec(blk, 1), _rows_spec(blk, 1),
             _rows_spec(blk, d), _rows_spec(blk, d)]
    return pl.pallas_call(
        fn,
        grid=(grid,),
        in_specs=specs,
        out_specs=[pl.BlockSpec((blk, 1), lambda i: (i, 0))] * 2 +
                  [pl.BlockSpec((blk, d), lambda i: (i, 0))] * 2,
        out_shape=[jax.ShapeDtypeStruct((nn, 1), F32)] * 2 +
                  [jax.ShapeDtypeStruct((nn, d), F32)] * 2,
    )(cnt_u.reshape(nn, 1), cnt_p.reshape(nn, 1), h1u, h1p)


def _tcC(acc1u, acc1p, dinvu, dinvp, b1, W2, blk):
    nn, d = acc1u.shape

    def fn(au_r, ap_r, du_r, dp_r, b1_r, w2_r, gu_r, gp_r):
        x1u = jnp.maximum(du_r[...] * au_r[...] + b1_r[...], 0.0)
        x1p = jnp.maximum(dp_r[...] * ap_r[...] + b1_r[...], 0.0)
        gu_r[...] = du_r[...] * _dot_t(x1u, w2_r[...])
        gp_r[...] = dp_r[...] * _dot_t(x1p, w2_r[...])

    specs = [_rows_spec(blk, d), _rows_spec(blk, d),
             _rows_spec(blk, 1), _rows_spec(blk, 1),
             _full_spec((1, d)), _full_spec(W2.shape)]
    args = (acc1u, acc1p, dinvu, dinvp, b1.reshape(1, d), W2)
    return _tc_call(fn, 2, blk, nn, d, args, specs)


def _tcD(acc2u, acc2p, dinvu, dinvp, b2, Wp1a, Wp1b, bp1, blk):
    nn, d = acc2u.shape

    def fn(au_r, ap_r, du_r, dp_r, b2_r, wa_r, wb_r, bp1_r, ua_r, pb_r):
        x2u = du_r[...] * au_r[...] + b2_r[...]
        x2p = dp_r[...] * ap_r[...] + b2_r[...]
        ua_r[...] = _dot_t(x2u, wa_r[...]) + bp1_r[...]
        pb_r[...] = _dot_t(x2p, wb_r[...])

    specs = [_rows_spec(blk, d), _rows_spec(blk, d),
             _rows_spec(blk, 1), _rows_spec(blk, 1),
             _full_spec((1, d)), _full_spec(Wp1a.shape),
             _full_spec(Wp1b.shape), _full_spec((1, d))]
    args = (acc2u, acc2p, dinvu, dinvp, b2.reshape(1, d), Wp1a, Wp1b,
            bp1.reshape(1, d))
    return _tc_call(fn, 2, blk, nn, d, args, specs)


def kernel(edge_index, user_features, product_features, user_emb, product_emb,
           W_uf, b_uf, W_pf, b_pf, W1, b1, W2, b2, Wp1, bp1, Wp2, bp2):
    nn = user_features.shape[0]
    d = W1.shape[0]
    e = edge_index.shape[1]
    r = e // _LANE
    blk = 5000

    u2d = edge_index[0].reshape(r, _LANE)
    p2d = edge_index[1].reshape(r, _LANE)

    pad = ((nn + 127) // 128) * 128
    cnt_u, cnt_p = _run_hist(u2d, p2d, nn)
    h1u, h1p = _tcA(user_features, product_features, user_emb, product_emb,
                    W_uf, b_uf, W_pf, b_pf, W1, blk)
    dinvu, dinvp, g1u, g1p = _tcB(cnt_u, cnt_p, h1u, h1p, blk)
    acc1u, acc1p = _run_conv(u2d, p2d, g1u, g1p, pad)
    g2u, g2p = _tcC(acc1u, acc1p, dinvu, dinvp, b1, W2, blk)
    acc2u, acc2p = _run_conv(u2d, p2d, g2u, g2p, pad)
    ua, pb = _tcD(acc2u, acc2p, dinvu, dinvp, b2,
                  Wp1[:, :d], Wp1[:, d:], bp1, blk)
    pred = _run_pred(u2d, p2d, ua, pb, Wp2.reshape(d), bp2)
    return pred


# speedup vs baseline: 40.1342x; 1.1440x over previous
import functools

import jax
import jax.numpy as jnp
from jax import lax
from jax.experimental import pallas as pl
from jax.experimental.pallas import tpu as pltpu
from jax.experimental.pallas import tpu_sc as plsc

F32 = jnp.float32
_MESH = plsc.VectorSubcoreMesh(core_axis_name="c", subcore_axis_name="s")
_SC_PARAMS = pltpu.CompilerParams(use_tc_tiling_on_sc=False,
                                  needs_layout_passes=False)
_NSUB = 16
_NCORE = 2
_LANE = 128
_CH = 13


def _row_partition(nrows, nworkers, w):
    per = nrows // nworkers
    extra = nrows % nworkers
    base = w * per + jnp.minimum(w, extra)
    count = per + jnp.where(w < extra, 1, 0)
    return base, count


def _hist_kernel(pad, nn, r):
    chunk = pad // _NSUB
    tail_valid = nn - (_NSUB - 1) * chunk

    def body(u2d, p2d, cnt_u, cnt_p, idx_v, ones_v, stage, acc, ssem):
        c = lax.axis_index("c")
        s = lax.axis_index("s")
        for i in range(8):
            ones_v[pl.ds(i * 16, 16)] = jnp.ones((16,), F32)
        zero16 = jnp.zeros((16,), F32)
        for i in range(chunk // 16):
            stage[pl.ds(i * 16, 16)] = zero16
        pltpu.sync_copy(stage, acc.at[pl.ds(s * chunk, chunk)])
        plsc.subcore_barrier()
        base, count = _row_partition(r, _NSUB, s)
        nch = count // _CH
        rem = count - nch * _CH

        def run(e2d):
            def chunk_body(jc, carry):
                r0 = base + jc * _CH
                pltpu.sync_copy(e2d.at[pl.ds(r0, _CH)], idx_v)
                descs = [pltpu.async_copy(ones_v, acc.at[idx_v.at[k]],
                                          ssem, add=True)
                         for k in range(_CH)]
                for de in descs:
                    de.wait()
                return carry

            lax.fori_loop(0, nch, chunk_body, 0)

            def tail_body(i, carry):
                rr = base + nch * _CH + i
                pltpu.sync_copy(e2d.at[pl.ds(rr, 1)], idx_v.at[pl.ds(0, 1)])
                pltpu.sync_copy(ones_v, acc.at[idx_v.at[0]], add=True)
                return carry

            lax.fori_loop(0, rem, tail_body, 0)

        @pl.when(c == 0)
        def _():
            run(u2d)

        @pl.when(c == 1)
        def _():
            run(p2d)

        plsc.subcore_barrier()
        pltpu.sync_copy(acc.at[pl.ds(s * chunk, chunk)], stage)

        def drain(cnt):
            @pl.when(s < _NSUB - 1)
            def _():
                pltpu.sync_copy(stage, cnt.at[pl.ds(s * chunk, chunk)])

            @pl.when(s == _NSUB - 1)
            def _():
                pltpu.sync_copy(stage.at[pl.ds(0, tail_valid)],
                                cnt.at[pl.ds(s * chunk, tail_valid)])

        @pl.when(c == 0)
        def _():
            drain(cnt_u)

        @pl.when(c == 1)
        def _():
            drain(cnt_p)

    return body


def _run_hist(u2d, p2d, nn):
    r = u2d.shape[0]
    pad = ((nn + 127) // 128) * 128
    cu, cp = pl.kernel(
        _hist_kernel(pad, nn, r),
        out_type=[jax.ShapeDtypeStruct((nn,), F32),
                  jax.ShapeDtypeStruct((nn,), F32)],
        mesh=_MESH,
        compiler_params=_SC_PARAMS,
        scratch_types=[pltpu.VMEM((_CH, _LANE), jnp.int32),
                       pltpu.VMEM((_LANE,), F32),
                       pltpu.VMEM((pad // _NSUB,), F32),
                       pltpu.VMEM_SHARED((pad,), F32),
                       pltpu.SemaphoreType.DMA],
    )(u2d, p2d)
    return cu, cp


_SUBROWS = 112
_NBUF = 3
_LOOK = 1


def _conv_kernel(pad, nn, d, r):
    chunk = pad // _NSUB
    nit = chunk // _SUBROWS
    tail_rows = nn - (_NSUB - 1) * chunk
    tail_full = tail_rows // _SUBROWS
    tail_part = tail_rows - tail_full * _SUBROWS

    def body(u2d, p2d, g_u, g_p, acc_u_out, acc_p_out,
             idxu_blk, idxp_blk, rows0, rows1, rows2, acc,
             gsem0, gsem1, gsem2, ssem0, ssem1, ssem2, isem0, isem1):
        c = lax.axis_index("c")
        s = lax.axis_index("s")
        rows = (rows0, rows1, rows2)
        gsem = (gsem0, gsem1, gsem2)
        ssem = (ssem0, ssem1, ssem2)

        def blockcopy(src, dst):
            def tile(off, n, via):
                pltpu.sync_copy(src.at[pl.ds(off, n)], via)
                pltpu.sync_copy(via, dst.at[pl.ds(off, n)])

            for t in range(nit):
                off = s * chunk + t * _SUBROWS
                if t < tail_full:
                    tile(off, _SUBROWS, rows0.at[pl.ds(0, _SUBROWS)])
                else:
                    @pl.when(s < _NSUB - 1)
                    def _():
                        tile(off, _SUBROWS, rows0.at[pl.ds(0, _SUBROWS)])

                    if t == tail_full and tail_part > 0:
                        @pl.when(s == _NSUB - 1)
                        def _():
                            tile(off, tail_part,
                                 rows0.at[pl.ds(0, tail_part)])

        base, count = _row_partition(r, _NSUB, s)
        nch = count // _CH
        rem = count - nch * _CH

        def run(g_self, g_src, idx_src, idx_dst, out_ref):
            blockcopy(g_self, acc)
            plsc.subcore_barrier()

            def chunk_body(jc, carry):
                r0 = base + jc * _CH
                ci = pltpu.async_copy(u2d.at[pl.ds(r0, _CH)], idxu_blk, isem0)
                cj = pltpu.async_copy(p2d.at[pl.ds(r0, _CH)], idxp_blk, isem1)
                ci.wait()
                cj.wait()
                g_desc = {}
                s_desc = {}
                for j in range(_LOOK):
                    g_desc[j] = pltpu.async_copy(
                        g_src.at[idx_src.at[j]], rows[j % _NBUF],
                        gsem[j % _NBUF])
                for k in range(_CH):
                    b = k % _NBUF
                    j = k + _LOOK
                    if j < _CH:
                        if j - _NBUF >= 0:
                            s_desc[j - _NBUF].wait()
                        g_desc[j] = pltpu.async_copy(
                            g_src.at[idx_src.at[j]], rows[j % _NBUF],
                            gsem[j % _NBUF])
                    g_desc[k].wait()
                    s_desc[k] = pltpu.async_copy(
                        rows[b], acc.at[idx_dst.at[k]], ssem[b], add=True)
                for k in range(max(0, _CH - _NBUF), _CH):
                    s_desc[k].wait()
                return carry

            lax.fori_loop(0, nch, chunk_body, 0)

            def tail_body(i, carry):
                rr = base + nch * _CH + i
                pltpu.sync_copy(u2d.at[pl.ds(rr, 1)], idxu_blk.at[pl.ds(0, 1)])
                pltpu.sync_copy(p2d.at[pl.ds(rr, 1)], idxp_blk.at[pl.ds(0, 1)])
                pltpu.async_copy(g_src.at[idx_src.at[0]], rows[0],
                                 gsem[0]).wait()
                pltpu.sync_copy(rows[0], acc.at[idx_dst.at[0]], add=True)
                return carry

            lax.fori_loop(0, rem, tail_body, 0)
            plsc.subcore_barrier()
            blockcopy(acc, out_ref)

        @pl.when(c == 0)
        def _():
            run(g_u, g_p, idxp_blk, idxu_blk, acc_u_out)

        @pl.when(c == 1)
        def _():
            run(g_p, g_u, idxu_blk, idxp_blk, acc_p_out)

    return body


def _run_conv(u2d, p2d, g_u, g_p, pad):
    nn, d = g_u.shape
    r = u2d.shape[0]
    au, ap = pl.kernel(
        _conv_kernel(pad, nn, d, r),
        out_type=[jax.ShapeDtypeStruct((nn, d), F32),
                  jax.ShapeDtypeStruct((nn, d), F32)],
        mesh=_MESH,
        compiler_params=_SC_PARAMS,
        scratch_types=[pltpu.VMEM((_CH, _LANE), jnp.int32),
                       pltpu.VMEM((_CH, _LANE), jnp.int32)] +
                      [pltpu.VMEM((_LANE, d), F32)] * _NBUF +
                      [pltpu.VMEM_SHARED((pad, d), F32)] +
                      [pltpu.SemaphoreType.DMA] * (2 * _NBUF + 2),
    )(u2d, p2d, g_u, g_p)
    return au, ap


_PCH = 8


def _pred_kernel(d, r):
    nb = d // 16

    def body(u2d, p2d, ua, pb, w2_hbm, bp2_hbm, pred,
             idx_u, idx_p, abuf0, abuf1, bbuf0, bbuf1, wbuf, bpbuf,
             obuf, tbuf, asem0, asem1, bsem0, bsem1, isem0, isem1):
        c = lax.axis_index("c")
        s = lax.axis_index("s")
        abuf = (abuf0, abuf1)
        bbuf = (bbuf0, bbuf1)
        asem = (asem0, asem1)
        bsem = (bsem0, bsem1)
        pltpu.sync_copy(w2_hbm, wbuf)
        pltpu.sync_copy(bp2_hbm, bpbuf)
        wv = [wbuf[pl.ds(k * 16, 16)] for k in range(nb)]
        bpv = bpbuf[pl.ds(0, 16)]
        lane = lax.iota(jnp.int32, 16)
        w = s * _NCORE + c
        base, count = _row_partition(r, _NSUB * _NCORE, w)
        nch = count // _PCH
        rem = count - nch * _PCH

        def compute_row(ab, bb, k):
            def group(g, carry2):
                for jj in range(16):
                    j = g * 16 + jj
                    t = None
                    for kk in range(nb):
                        a = ab[j, pl.ds(kk * 16, 16)]
                        b = bb[j, pl.ds(kk * 16, 16)]
                        part = jnp.maximum(a + b, 0.0) * wv[kk]
                        t = part if t is None else t + part
                    plsc.store_scatter(tbuf, [lane * 16 + jj], t)
                t8 = [tbuf[pl.ds(kk * 32, 16)] + tbuf[pl.ds(kk * 32 + 16, 16)]
                      for kk in range(8)]
                t4 = [t8[2 * kk] + t8[2 * kk + 1] for kk in range(4)]
                t2 = [t4[0] + t4[1], t4[2] + t4[3]]
                obuf[pl.ds(k * _LANE + g * 16, 16)] = bpv + (t2[0] + t2[1])
                return carry2

            lax.fori_loop(0, _LANE // 16, group, 0)

        def chunk_body(jc, carry):
            r0 = base + jc * _PCH
            ci = pltpu.async_copy(u2d.at[pl.ds(r0, _PCH)], idx_u, isem0)
            cj = pltpu.async_copy(p2d.at[pl.ds(r0, _PCH)], idx_p, isem1)
            ci.wait()
            cj.wait()
            a_desc = {0: pltpu.async_copy(ua.at[idx_u.at[0]], abuf[0],
                                          asem[0])}
            b_desc = {0: pltpu.async_copy(pb.at[idx_p.at[0]], bbuf[0],
                                          bsem[0])}
            for k in range(_PCH):
                b = k & 1
                if k + 1 < _PCH:
                    a_desc[k + 1] = pltpu.async_copy(
                        ua.at[idx_u.at[k + 1]], abuf[1 - b], asem[1 - b])
                    b_desc[k + 1] = pltpu.async_copy(
                        pb.at[idx_p.at[k + 1]], bbuf[1 - b], bsem[1 - b])
                a_desc[k].wait()
                b_desc[k].wait()
                compute_row(abuf[b], bbuf[b], k)
            pltpu.sync_copy(obuf,
                            pred.at[pl.ds(r0 * _LANE, _PCH * _LANE)])
            return carry

        lax.fori_loop(0, nch, chunk_body, 0)

        def tail_body(i, carry):
            rr = base + nch * _PCH + i
            pltpu.sync_copy(u2d.at[pl.ds(rr, 1)], idx_u.at[pl.ds(0, 1)])
            pltpu.sync_copy(p2d.at[pl.ds(rr, 1)], idx_p.at[pl.ds(0, 1)])
            cp_a = pltpu.async_copy(ua.at[idx_u.at[0]], abuf[0], asem[0])
            cp_b = pltpu.async_copy(pb.at[idx_p.at[0]], bbuf[0], bsem[0])
            cp_a.wait()
            cp_b.wait()
            compute_row(abuf[0], bbuf[0], 0)
            pltpu.sync_copy(obuf.at[pl.ds(0, _LANE)],
                            pred.at[pl.ds(rr * _LANE, _LANE)])
            return carry

        lax.fori_loop(0, rem, tail_body, 0)

    return body


def _run_pred(u2d, p2d, ua, pb, w2, bp2):
    nn, d = ua.shape
    r = u2d.shape[0]
    e = r * _LANE
    bp2_16 = jnp.broadcast_to(bp2.reshape(1), (16,)).astype(F32)
    return pl.kernel(
        _pred_kernel(d, r),
        out_type=jax.ShapeDtypeStruct((e,), F32),
        mesh=_MESH,
        compiler_params=_SC_PARAMS,
        scratch_types=[pltpu.VMEM((_PCH, _LANE), jnp.int32),
                       pltpu.VMEM((_PCH, _LANE), jnp.int32)] +
                      [pltpu.VMEM((_LANE, d), F32)] * 4 +
                      [pltpu.VMEM((d,), F32),
                       pltpu.VMEM((16,), F32),
                       pltpu.VMEM((_PCH * _LANE,), F32),
                       pltpu.VMEM((256,), F32)] +
                      [pltpu.SemaphoreType.DMA] * 6,
    )(u2d, p2d, ua, pb, w2, bp2_16)


def _dot_t(a, w):
    return lax.dot_general(a, w, (((1,), (1,)), ((), ())),
                           preferred_element_type=F32)


def _tc_call(fn, n_out, blk, nn, d, args, specs):
    grid = nn // blk
    return pl.pallas_call(
        fn,
        grid=(grid,),
        in_specs=specs,
        out_specs=[pl.BlockSpec((blk, d), lambda i: (i, 0))] * n_out,
        out_shape=[jax.ShapeDtypeStruct((nn, d), F32)] * n_out,
    )(*args)


def _rows_spec(blk, ncol):
    return pl.BlockSpec((blk, ncol), lambda i: (i, 0))


def _full_spec(shape):
    return pl.BlockSpec(shape, lambda i: tuple(0 for _ in shape))


def _tcA(uf, pf, ue, pe, W_uf, b_uf, W_pf, b_pf, W1, blk):
    nn, d = ue.shape

    def fn(uf_r, pf_r, ue_r, pe_r, wuf_r, buf_r, wpf_r, bpf_r, w1_r,
           h1u_r, h1p_r):
        xu = _dot_t(uf_r[...], wuf_r[...]) + buf_r[...] + ue_r[...]
        xp = _dot_t(pf_r[...], wpf_r[...]) + bpf_r[...] + pe_r[...]
        h1u_r[...] = _dot_t(xu, w1_r[...])
        h1p_r[...] = _dot_t(xp, w1_r[...])

    specs = [_rows_spec(blk, uf.shape[1]), _rows_spec(blk, pf.shape[1]),
             _rows_spec(blk, d), _rows_spec(blk, d),
             _full_spec(W_uf.shape), _full_spec((1, d)),
             _full_spec(W_pf.shape), _full_spec((1, d)),
             _full_spec(W1.shape)]
    args = (uf, pf, ue, pe, W_uf, b_uf.reshape(1, d), W_pf,
            b_pf.reshape(1, d), W1)
    return _tc_call(fn, 2, blk, nn, d, args, specs)


def _tcB(cnt_u, cnt_p, h1u, h1p, blk):
    nn, d = h1u.shape

    def fn(cu_r, cp_r, hu_r, hp_r, du_r, dp_r, gu_r, gp_r):
        du = (cu_r[...] + 1.0) ** -0.5
        dp = (cp_r[...] + 1.0) ** -0.5
        du_r[...] = du
        dp_r[...] = dp
        gu_r[...] = du * hu_r[...]
        gp_r[...] = dp * hp_r[...]

    grid = nn // blk
    specs = [_rows_spec(blk, 1), _rows_spec(blk, 1),
             _rows_spec(blk, d), _rows_spec(blk, d)]
    return pl.pallas_call(
        fn,
        grid=(grid,),
        in_specs=specs,
        out_specs=[pl.BlockSpec((blk, 1), lambda i: (i, 0))] * 2 +
                  [pl.BlockSpec((blk, d), lambda i: (i, 0))] * 2,
        out_shape=[jax.ShapeDtypeStruct((nn, 1), F32)] * 2 +
                  [jax.ShapeDtypeStruct((nn, d), F32)] * 2,
    )(cnt_u.reshape(nn, 1), cnt_p.reshape(nn, 1), h1u, h1p)


def _tcC(acc1u, acc1p, dinvu, dinvp, b1, W2, blk):
    nn, d = acc1u.shape

    def fn(au_r, ap_r, du_r, dp_r, b1_r, w2_r, gu_r, gp_r):
        x1u = jnp.maximum(du_r[...] * au_r[...] + b1_r[...], 0.0)
        x1p = jnp.maximum(dp_r[...] * ap_r[...] + b1_r[...], 0.0)
        gu_r[...] = du_r[...] * _dot_t(x1u, w2_r[...])
        gp_r[...] = dp_r[...] * _dot_t(x1p, w2_r[...])

    specs = [_rows_spec(blk, d), _rows_spec(blk, d),
             _rows_spec(blk, 1), _rows_spec(blk, 1),
             _full_spec((1, d)), _full_spec(W2.shape)]
    args = (acc1u, acc1p, dinvu, dinvp, b1.reshape(1, d), W2)
    return _tc_call(fn, 2, blk, nn, d, args, specs)


def _tcD(acc2u, acc2p, dinvu, dinvp, b2, Wp1a, Wp1b, bp1, blk):
    nn, d = acc2u.shape

    def fn(au_r, ap_r, du_r, dp_r, b2_r, wa_r, wb_r, bp1_r, ua_r, pb_r):
        x2u = du_r[...] * au_r[...] + b2_r[...]
        x2p = dp_r[...] * ap_r[...] + b2_r[...]
        ua_r[...] = _dot_t(x2u, wa_r[...]) + bp1_r[...]
        pb_r[...] = _dot_t(x2p, wb_r[...])

    specs = [_rows_spec(blk, d), _rows_spec(blk, d),
             _rows_spec(blk, 1), _rows_spec(blk, 1),
             _full_spec((1, d)), _full_spec(Wp1a.shape),
             _full_spec(Wp1b.shape), _full_spec((1, d))]
    args = (acc2u, acc2p, dinvu, dinvp, b2.reshape(1, d), Wp1a, Wp1b,
            bp1.reshape(1, d))
    return _tc_call(fn, 2, blk, nn, d, args, specs)


def kernel(edge_index, user_features, product_features, user_emb, product_emb,
           W_uf, b_uf, W_pf, b_pf, W1, b1, W2, b2, Wp1, bp1, Wp2, bp2):
    nn = user_features.shape[0]
    d = W1.shape[0]
    e = edge_index.shape[1]
    r = e // _LANE
    blk = 5000

    u2d = edge_index[0].reshape(r, _LANE)
    p2d = edge_index[1].reshape(r, _LANE)

    pad = ((nn + 127) // 128) * 128
    cnt_u, cnt_p = _run_hist(u2d, p2d, nn)
    h1u, h1p = _tcA(user_features, product_features, user_emb, product_emb,
                    W_uf, b_uf, W_pf, b_pf, W1, blk)
    dinvu, dinvp, g1u, g1p = _tcB(cnt_u, cnt_p, h1u, h1p, blk)
    acc1u, acc1p = _run_conv(u2d, p2d, g1u, g1p, pad)
    g2u, g2p = _tcC(acc1u, acc1p, dinvu, dinvp, b1, W2, blk)
    acc2u, acc2p = _run_conv(u2d, p2d, g2u, g2p, pad)
    ua, pb = _tcD(acc2u, acc2p, dinvu, dinvp, b2,
                  Wp1[:, :d], Wp1[:, d:], bp1, blk)
    pred = _run_pred(u2d, p2d, ua, pb, Wp2.reshape(d), bp2)
    return pred


# speedup vs baseline: 40.6944x; 1.0140x over previous
import functools

import jax
import jax.numpy as jnp
from jax import lax
from jax.experimental import pallas as pl
from jax.experimental.pallas import tpu as pltpu
from jax.experimental.pallas import tpu_sc as plsc

F32 = jnp.float32
_MESH = plsc.VectorSubcoreMesh(core_axis_name="c", subcore_axis_name="s")
_SC_PARAMS = pltpu.CompilerParams(use_tc_tiling_on_sc=False,
                                  needs_layout_passes=False)
_NSUB = 16
_NCORE = 2
_LANE = 128
_CH = 15


def _row_partition(nrows, nworkers, w):
    per = nrows // nworkers
    extra = nrows % nworkers
    base = w * per + jnp.minimum(w, extra)
    count = per + jnp.where(w < extra, 1, 0)
    return base, count


def _hist_kernel(pad, nn, r):
    chunk = pad // _NSUB
    tail_valid = nn - (_NSUB - 1) * chunk

    def body(u2d, p2d, cnt_u, cnt_p, idx_v, ones_v, stage, acc, ssem):
        c = lax.axis_index("c")
        s = lax.axis_index("s")
        for i in range(8):
            ones_v[pl.ds(i * 16, 16)] = jnp.ones((16,), F32)
        zero16 = jnp.zeros((16,), F32)
        for i in range(chunk // 16):
            stage[pl.ds(i * 16, 16)] = zero16
        pltpu.sync_copy(stage, acc.at[pl.ds(s * chunk, chunk)])
        plsc.subcore_barrier()
        base, count = _row_partition(r, _NSUB, s)
        nch = count // _CH
        rem = count - nch * _CH

        def run(e2d):
            def chunk_body(jc, carry):
                r0 = base + jc * _CH
                pltpu.sync_copy(e2d.at[pl.ds(r0, _CH)], idx_v)
                descs = [pltpu.async_copy(ones_v, acc.at[idx_v.at[k]],
                                          ssem, add=True)
                         for k in range(_CH)]
                for de in descs:
                    de.wait()
                return carry

            lax.fori_loop(0, nch, chunk_body, 0)

            def tail_body(i, carry):
                rr = base + nch * _CH + i
                pltpu.sync_copy(e2d.at[pl.ds(rr, 1)], idx_v.at[pl.ds(0, 1)])
                pltpu.sync_copy(ones_v, acc.at[idx_v.at[0]], add=True)
                return carry

            lax.fori_loop(0, rem, tail_body, 0)

        @pl.when(c == 0)
        def _():
            run(u2d)

        @pl.when(c == 1)
        def _():
            run(p2d)

        plsc.subcore_barrier()
        pltpu.sync_copy(acc.at[pl.ds(s * chunk, chunk)], stage)

        def drain(cnt):
            @pl.when(s < _NSUB - 1)
            def _():
                pltpu.sync_copy(stage, cnt.at[pl.ds(s * chunk, chunk)])

            @pl.when(s == _NSUB - 1)
            def _():
                pltpu.sync_copy(stage.at[pl.ds(0, tail_valid)],
                                cnt.at[pl.ds(s * chunk, tail_valid)])

        @pl.when(c == 0)
        def _():
            drain(cnt_u)

        @pl.when(c == 1)
        def _():
            drain(cnt_p)

    return body


def _run_hist(u2d, p2d, nn):
    r = u2d.shape[0]
    pad = ((nn + 127) // 128) * 128
    cu, cp = pl.kernel(
        _hist_kernel(pad, nn, r),
        out_type=[jax.ShapeDtypeStruct((nn,), F32),
                  jax.ShapeDtypeStruct((nn,), F32)],
        mesh=_MESH,
        compiler_params=_SC_PARAMS,
        scratch_types=[pltpu.VMEM((_CH, _LANE), jnp.int32),
                       pltpu.VMEM((_LANE,), F32),
                       pltpu.VMEM((pad // _NSUB,), F32),
                       pltpu.VMEM_SHARED((pad,), F32),
                       pltpu.SemaphoreType.DMA],
    )(u2d, p2d)
    return cu, cp


_SUBROWS = 112
_NBUF = 3
_LOOK = 2


def _conv_kernel(pad, nn, d, r):
    chunk = pad // _NSUB
    nit = chunk // _SUBROWS
    tail_rows = nn - (_NSUB - 1) * chunk
    tail_full = tail_rows // _SUBROWS
    tail_part = tail_rows - tail_full * _SUBROWS

    def body(u2d, p2d, g_u, g_p, acc_u_out, acc_p_out,
             idxu_blk, idxp_blk, rows0, rows1, rows2, acc,
             gsem0, gsem1, gsem2, ssem0, ssem1, ssem2, isem0, isem1):
        c = lax.axis_index("c")
        s = lax.axis_index("s")
        rows = (rows0, rows1, rows2)
        gsem = (gsem0, gsem1, gsem2)
        ssem = (ssem0, ssem1, ssem2)

        def blockcopy(src, dst):
            def tile(off, n, via):
                pltpu.sync_copy(src.at[pl.ds(off, n)], via)
                pltpu.sync_copy(via, dst.at[pl.ds(off, n)])

            for t in range(nit):
                off = s * chunk + t * _SUBROWS
                if t < tail_full:
                    tile(off, _SUBROWS, rows0.at[pl.ds(0, _SUBROWS)])
                else:
                    @pl.when(s < _NSUB - 1)
                    def _():
                        tile(off, _SUBROWS, rows0.at[pl.ds(0, _SUBROWS)])

                    if t == tail_full and tail_part > 0:
                        @pl.when(s == _NSUB - 1)
                        def _():
                            tile(off, tail_part,
                                 rows0.at[pl.ds(0, tail_part)])

        base, count = _row_partition(r, _NSUB, s)
        nch = count // _CH
        rem = count - nch * _CH

        def run(g_self, g_src, idx_src, idx_dst, out_ref):
            blockcopy(g_self, acc)
            plsc.subcore_barrier()

            def chunk_body(jc, carry):
                r0 = base + jc * _CH
                ci = pltpu.async_copy(u2d.at[pl.ds(r0, _CH)], idxu_blk, isem0)
                cj = pltpu.async_copy(p2d.at[pl.ds(r0, _CH)], idxp_blk, isem1)
                ci.wait()
                cj.wait()
                g_desc = {}
                s_desc = {}
                for j in range(_LOOK):
                    g_desc[j] = pltpu.async_copy(
                        g_src.at[idx_src.at[j]], rows[j % _NBUF],
                        gsem[j % _NBUF])
                for k in range(_CH):
                    b = k % _NBUF
                    j = k + _LOOK
                    if j < _CH:
                        if j - _NBUF >= 0:
                            s_desc[j - _NBUF].wait()
                        g_desc[j] = pltpu.async_copy(
                            g_src.at[idx_src.at[j]], rows[j % _NBUF],
                            gsem[j % _NBUF])
                    g_desc[k].wait()
                    s_desc[k] = pltpu.async_copy(
                        rows[b], acc.at[idx_dst.at[k]], ssem[b], add=True)
                for k in range(max(0, _CH - _NBUF), _CH):
                    s_desc[k].wait()
                return carry

            lax.fori_loop(0, nch, chunk_body, 0)

            def tail_body(i, carry):
                rr = base + nch * _CH + i
                pltpu.sync_copy(u2d.at[pl.ds(rr, 1)], idxu_blk.at[pl.ds(0, 1)])
                pltpu.sync_copy(p2d.at[pl.ds(rr, 1)], idxp_blk.at[pl.ds(0, 1)])
                pltpu.async_copy(g_src.at[idx_src.at[0]], rows[0],
                                 gsem[0]).wait()
                pltpu.sync_copy(rows[0], acc.at[idx_dst.at[0]], add=True)
                return carry

            lax.fori_loop(0, rem, tail_body, 0)
            plsc.subcore_barrier()
            blockcopy(acc, out_ref)

        @pl.when(c == 0)
        def _():
            run(g_u, g_p, idxp_blk, idxu_blk, acc_u_out)

        @pl.when(c == 1)
        def _():
            run(g_p, g_u, idxu_blk, idxp_blk, acc_p_out)

    return body


def _run_conv(u2d, p2d, g_u, g_p, pad):
    nn, d = g_u.shape
    r = u2d.shape[0]
    au, ap = pl.kernel(
        _conv_kernel(pad, nn, d, r),
        out_type=[jax.ShapeDtypeStruct((nn, d), F32),
                  jax.ShapeDtypeStruct((nn, d), F32)],
        mesh=_MESH,
        compiler_params=_SC_PARAMS,
        scratch_types=[pltpu.VMEM((_CH, _LANE), jnp.int32),
                       pltpu.VMEM((_CH, _LANE), jnp.int32)] +
                      [pltpu.VMEM((_LANE, d), F32)] * _NBUF +
                      [pltpu.VMEM_SHARED((pad, d), F32)] +
                      [pltpu.SemaphoreType.DMA] * (2 * _NBUF + 2),
    )(u2d, p2d, g_u, g_p)
    return au, ap


_PCH = 8


def _pred_kernel(d, r):
    nb = d // 16

    def body(u2d, p2d, ua, pb, w2_hbm, bp2_hbm, pred,
             idx_u, idx_p, abuf0, abuf1, bbuf0, bbuf1, wbuf, bpbuf,
             obuf, tbuf, asem0, asem1, bsem0, bsem1, isem0, isem1):
        c = lax.axis_index("c")
        s = lax.axis_index("s")
        abuf = (abuf0, abuf1)
        bbuf = (bbuf0, bbuf1)
        asem = (asem0, asem1)
        bsem = (bsem0, bsem1)
        pltpu.sync_copy(w2_hbm, wbuf)
        pltpu.sync_copy(bp2_hbm, bpbuf)
        wv = [wbuf[pl.ds(k * 16, 16)] for k in range(nb)]
        bpv = bpbuf[pl.ds(0, 16)]
        lane = lax.iota(jnp.int32, 16)
        w = s * _NCORE + c
        base, count = _row_partition(r, _NSUB * _NCORE, w)
        nch = count // _PCH
        rem = count - nch * _PCH

        def compute_row(ab, bb, k):
            def group(g, carry2):
                for jj in range(16):
                    j = g * 16 + jj
                    t = None
                    for kk in range(nb):
                        a = ab[j, pl.ds(kk * 16, 16)]
                        b = bb[j, pl.ds(kk * 16, 16)]
                        part = jnp.maximum(a + b, 0.0) * wv[kk]
                        t = part if t is None else t + part
                    plsc.store_scatter(tbuf, [lane * 16 + jj], t)
                t8 = [tbuf[pl.ds(kk * 32, 16)] + tbuf[pl.ds(kk * 32 + 16, 16)]
                      for kk in range(8)]
                t4 = [t8[2 * kk] + t8[2 * kk + 1] for kk in range(4)]
                t2 = [t4[0] + t4[1], t4[2] + t4[3]]
                obuf[pl.ds(k * _LANE + g * 16, 16)] = bpv + (t2[0] + t2[1])
                return carry2

            lax.fori_loop(0, _LANE // 16, group, 0)

        def chunk_body(jc, carry):
            r0 = base + jc * _PCH
            ci = pltpu.async_copy(u2d.at[pl.ds(r0, _PCH)], idx_u, isem0)
            cj = pltpu.async_copy(p2d.at[pl.ds(r0, _PCH)], idx_p, isem1)
            ci.wait()
            cj.wait()
            a_desc = {0: pltpu.async_copy(ua.at[idx_u.at[0]], abuf[0],
                                          asem[0])}
            b_desc = {0: pltpu.async_copy(pb.at[idx_p.at[0]], bbuf[0],
                                          bsem[0])}
            for k in range(_PCH):
                b = k & 1
                if k + 1 < _PCH:
                    a_desc[k + 1] = pltpu.async_copy(
                        ua.at[idx_u.at[k + 1]], abuf[1 - b], asem[1 - b])
                    b_desc[k + 1] = pltpu.async_copy(
                        pb.at[idx_p.at[k + 1]], bbuf[1 - b], bsem[1 - b])
                a_desc[k].wait()
                b_desc[k].wait()
                compute_row(abuf[b], bbuf[b], k)
            pltpu.sync_copy(obuf,
                            pred.at[pl.ds(r0 * _LANE, _PCH * _LANE)])
            return carry

        lax.fori_loop(0, nch, chunk_body, 0)

        def tail_body(i, carry):
            rr = base + nch * _PCH + i
            pltpu.sync_copy(u2d.at[pl.ds(rr, 1)], idx_u.at[pl.ds(0, 1)])
            pltpu.sync_copy(p2d.at[pl.ds(rr, 1)], idx_p.at[pl.ds(0, 1)])
            cp_a = pltpu.async_copy(ua.at[idx_u.at[0]], abuf[0], asem[0])
            cp_b = pltpu.async_copy(pb.at[idx_p.at[0]], bbuf[0], bsem[0])
            cp_a.wait()
            cp_b.wait()
            compute_row(abuf[0], bbuf[0], 0)
            pltpu.sync_copy(obuf.at[pl.ds(0, _LANE)],
                            pred.at[pl.ds(rr * _LANE, _LANE)])
            return carry

        lax.fori_loop(0, rem, tail_body, 0)

    return body


def _run_pred(u2d, p2d, ua, pb, w2, bp2):
    nn, d = ua.shape
    r = u2d.shape[0]
    e = r * _LANE
    bp2_16 = jnp.broadcast_to(bp2.reshape(1), (16,)).astype(F32)
    return pl.kernel(
        _pred_kernel(d, r),
        out_type=jax.ShapeDtypeStruct((e,), F32),
        mesh=_MESH,
        compiler_params=_SC_PARAMS,
        scratch_types=[pltpu.VMEM((_PCH, _LANE), jnp.int32),
                       pltpu.VMEM((_PCH, _LANE), jnp.int32)] +
                      [pltpu.VMEM((_LANE, d), F32)] * 4 +
                      [pltpu.VMEM((d,), F32),
                       pltpu.VMEM((16,), F32),
                       pltpu.VMEM((_PCH * _LANE,), F32),
                       pltpu.VMEM((256,), F32)] +
                      [pltpu.SemaphoreType.DMA] * 6,
    )(u2d, p2d, ua, pb, w2, bp2_16)


def _dot_t(a, w):
    return lax.dot_general(a, w, (((1,), (1,)), ((), ())),
                           preferred_element_type=F32)


def _tc_call(fn, n_out, blk, nn, d, args, specs):
    grid = nn // blk
    return pl.pallas_call(
        fn,
        grid=(grid,),
        in_specs=specs,
        out_specs=[pl.BlockSpec((blk, d), lambda i: (i, 0))] * n_out,
        out_shape=[jax.ShapeDtypeStruct((nn, d), F32)] * n_out,
    )(*args)


def _rows_spec(blk, ncol):
    return pl.BlockSpec((blk, ncol), lambda i: (i, 0))


def _full_spec(shape):
    return pl.BlockSpec(shape, lambda i: tuple(0 for _ in shape))


def _tcA(uf, pf, ue, pe, W_uf, b_uf, W_pf, b_pf, W1, blk):
    nn, d = ue.shape

    def fn(uf_r, pf_r, ue_r, pe_r, wuf_r, buf_r, wpf_r, bpf_r, w1_r,
           h1u_r, h1p_r):
        xu = _dot_t(uf_r[...], wuf_r[...]) + buf_r[...] + ue_r[...]
        xp = _dot_t(pf_r[...], wpf_r[...]) + bpf_r[...] + pe_r[...]
        h1u_r[...] = _dot_t(xu, w1_r[...])
        h1p_r[...] = _dot_t(xp, w1_r[...])

    specs = [_rows_spec(blk, uf.shape[1]), _rows_spec(blk, pf.shape[1]),
             _rows_spec(blk, d), _rows_spec(blk, d),
             _full_spec(W_uf.shape), _full_spec((1, d)),
             _full_spec(W_pf.shape), _full_spec((1, d)),
             _full_spec(W1.shape)]
    args = (uf, pf, ue, pe, W_uf, b_uf.reshape(1, d), W_pf,
            b_pf.reshape(1, d), W1)
    return _tc_call(fn, 2, blk, nn, d, args, specs)


def _tcB(cnt_u, cnt_p, h1u, h1p, blk):
    nn, d = h1u.shape

    def fn(cu_r, cp_r, hu_r, hp_r, du_r, dp_r, gu_r, gp_r):
        du = (cu_r[...] + 1.0) ** -0.5
        dp = (cp_r[...] + 1.0) ** -0.5
        du_r[...] = du
        dp_r[...] = dp
        gu_r[...] = du * hu_r[...]
        gp_r[...] = dp * hp_r[...]

    grid = nn // blk
    specs = [_rows_spec(blk, 1), _rows_spec(blk, 1),
             _rows_spec(blk, d), _rows_spec(blk, d)]
    return pl.pallas_call(
        fn,
        grid=(grid,),
        in_specs=specs,
        out_specs=[pl.BlockSpec((blk, 1), lambda i: (i, 0))] * 2 +
                  [pl.BlockSpec((blk, d), lambda i: (i, 0))] * 2,
        out_shape=[jax.ShapeDtypeStruct((nn, 1), F32)] * 2 +
                  [jax.ShapeDtypeStruct((nn, d), F32)] * 2,
    )(cnt_u.reshape(nn, 1), cnt_p.reshape(nn, 1), h1u, h1p)


def _tcC(acc1u, acc1p, dinvu, dinvp, b1, W2, blk):
    nn, d = acc1u.shape

    def fn(au_r, ap_r, du_r, dp_r, b1_r, w2_r, gu_r, gp_r):
        x1u = jnp.maximum(du_r[...] * au_r[...] + b1_r[...], 0.0)
        x1p = jnp.maximum(dp_r[...] * ap_r[...] + b1_r[...], 0.0)
        gu_r[...] = du_r[...] * _dot_t(x1u, w2_r[...])
        gp_r[...] = dp_r[...] * _dot_t(x1p, w2_r[...])

    specs = [_rows_spec(blk, d), _rows_spec(blk, d),
             _rows_spec(blk, 1), _rows_spec(blk, 1),
             _full_spec((1, d)), _full_spec(W2.shape)]
    args = (acc1u, acc1p, dinvu, dinvp, b1.reshape(1, d), W2)
    return _tc_call(fn, 2, blk, nn, d, args, specs)


def _tcD(acc2u, acc2p, dinvu, dinvp, b2, Wp1a, Wp1b, bp1, blk):
    nn, d = acc2u.shape

    def fn(au_r, ap_r, du_r, dp_r, b2_r, wa_r, wb_r, bp1_r, ua_r, pb_r):
        x2u = du_r[...] * au_r[...] + b2_r[...]
        x2p = dp_r[...] * ap_r[...] + b2_r[...]
        ua_r[...] = _dot_t(x2u, wa_r[...]) + bp1_r[...]
        pb_r[...] = _dot_t(x2p, wb_r[...])

    specs = [_rows_spec(blk, d), _rows_spec(blk, d),
             _rows_spec(blk, 1), _rows_spec(blk, 1),
             _full_spec((1, d)), _full_spec(Wp1a.shape),
             _full_spec(Wp1b.shape), _full_spec((1, d))]
    args = (acc2u, acc2p, dinvu, dinvp, b2.reshape(1, d), Wp1a, Wp1b,
            bp1.reshape(1, d))
    return _tc_call(fn, 2, blk, nn, d, args, specs)


def kernel(edge_index, user_features, product_features, user_emb, product_emb,
           W_uf, b_uf, W_pf, b_pf, W1, b1, W2, b2, Wp1, bp1, Wp2, bp2):
    nn = user_features.shape[0]
    d = W1.shape[0]
    e = edge_index.shape[1]
    r = e // _LANE
    blk = 5000

    u2d = edge_index[0].reshape(r, _LANE)
    p2d = edge_index[1].reshape(r, _LANE)

    pad = ((nn + 127) // 128) * 128
    cnt_u, cnt_p = _run_hist(u2d, p2d, nn)
    h1u, h1p = _tcA(user_features, product_features, user_emb, product_emb,
                    W_uf, b_uf, W_pf, b_pf, W1, blk)
    dinvu, dinvp, g1u, g1p = _tcB(cnt_u, cnt_p, h1u, h1p, blk)
    acc1u, acc1p = _run_conv(u2d, p2d, g1u, g1p, pad)
    g2u, g2p = _tcC(acc1u, acc1p, dinvu, dinvp, b1, W2, blk)
    acc2u, acc2p = _run_conv(u2d, p2d, g2u, g2p, pad)
    ua, pb = _tcD(acc2u, acc2p, dinvu, dinvp, b2,
                  Wp1[:, :d], Wp1[:, d:], bp1, blk)
    pred = _run_pred(u2d, p2d, ua, pb, Wp2.reshape(d), bp2)
    return pred
